# baseline (speedup 1.0000x reference)
import functools

import jax
import jax.numpy as jnp
from jax import lax
from jax.experimental import pallas as pl
from jax.experimental.pallas import tpu as pltpu
from jax.experimental.pallas import tpu_sc as plsc

N = 10000
B = 100
NPER = 100
D = 256
H = 128
E = 160000
NC = 2
NS = 16
K = 64
NB = 4
G = 8
NG = 20
NP = NG // 2
CH = G * NG
EPT = CH * K
EP = NS * EPT
ZPT = N // NS
RB = 2000


def _mm_body(x_ref, w_ref, b_ref, anch_ref, feat_ref, aout_ref):
    i = pl.program_id(0)
    xb = x_ref[...]
    row = lax.broadcasted_iota(jnp.int32, (RB, 1), 0) + i * RB
    xb = jnp.where(row % NPER == 0, 0.0, xb)
    f = jnp.dot(xb, w_ref[...], preferred_element_type=jnp.float32)
    feat_ref[0] = f[:, :H]
    feat_ref[1] = f[:, H:]

    @pl.when(i == 0)
    def _():
        aout_ref[...] = (
            jnp.dot(anch_ref[...], w_ref[...], preferred_element_type=jnp.float32)
            + b_ref[...]
        )


def _phase_a(x, W, b2, anchors):
    return pl.pallas_call(
        _mm_body,
        grid=(N // RB,),
        in_specs=[
            pl.BlockSpec((RB, D), lambda i: (i, 0)),
            pl.BlockSpec((D, D), lambda i: (0, 0)),
            pl.BlockSpec((1, D), lambda i: (0, 0)),
            pl.BlockSpec((B, D), lambda i: (0, 0)),
        ],
        out_specs=[
            pl.BlockSpec((2, RB, H), lambda i: (0, i, 0)),
            pl.BlockSpec((B, D), lambda i: (0, 0)),
        ],
        out_shape=[
            jax.ShapeDtypeStruct((2, N, H), jnp.float32),
            jax.ShapeDtypeStruct((B, D), jnp.float32),
        ],
    )(x, W, b2, anchors)


def _sc_body(feat_hbm, src_hbm, dst_hbm, w_hbm, b_hbm, a_hbm, z_hbm, out_hbm,
             es0, ed0, ew0, es1, ed1, ew1, buf0, buf1, buf2, sbuf0, sbuf1,
             b_v, a_v, prow,
             gsem0, gsem1, gsem2, ssem0, ssem1,
             esem0, esem1, sh_h):
    half = lax.axis_index("c")
    t = lax.axis_index("s")
    bufs = (buf0, buf1, buf2)
    sbufs = (sbuf0, sbuf1)
    gsems = (gsem0, gsem1, gsem2)
    ssems = (ssem0, ssem1)
    egs = ((es0, ed0, ew0, esem0), (es1, ed1, ew1, esem1))

    off = half * N

    def _stage_copies(gi, eg):
        es, ed, ew, esem = eg
        gsl = pl.ds(gi * G, G)
        return (pltpu.make_async_copy(src_hbm.at[t, gsl], es, esem),
                pltpu.make_async_copy(dst_hbm.at[t, gsl], ed, esem),
                pltpu.make_async_copy(w_hbm.at[t, gsl], ew, esem))

    def _stage_start(gi, eg):
        for cp in _stage_copies(gi, eg):
            cp.start()

    def _stage_finish(gi, eg):
        for cp in _stage_copies(gi, eg):
            cp.wait()
        es = eg[0]
        for j in range(G):
            for c in range(K // 16):
                sl = pl.ds(c * 16, 16)
                es[j, sl] = es[j, sl] + off

    def _gather(j, eg):
        p = j % 3
        return pltpu.make_async_copy(feat_hbm.at[eg[0].at[j]], bufs[p],
                                     gsems[p])

    def _scatter(j, eg):
        q = j % 2
        return pltpu.make_async_copy(sbufs[q], sh_h.at[eg[1].at[j]], ssems[q])

    def _scale(j, eg):
        buf = bufs[j % 3]
        sbuf = sbufs[j % 2]
        ew = eg[2]

        def _body(k, _):
            wk = plsc.load_gather(
                ew, [jnp.full((16,), j, jnp.int32),
                     jnp.full((16,), k, jnp.int32)])
            for c in range(H // 16):
                sl = pl.ds(c * 16, 16)
                sbuf[k, sl] = buf[k, sl] * wk
            return 0

        lax.fori_loop(0, K, _body, 0, unroll=4)

    def _process_group(eg):
        for jj in range(2):
            _gather(jj, eg).start()
        for j in range(G):
            if j + 2 < G:
                _gather(j + 2, eg).start()
            _gather(j, eg).wait()
            if j >= 2:
                _scatter(j - 2, eg).wait()
            _scale(j, eg)
            _scatter(j, eg).start(add=True)
        for j in range(G - 2, G):
            _scatter(j, eg).wait()

    _stage_start(0, egs[0])
    pltpu.sync_copy(b_hbm.at[half], b_v)
    pltpu.sync_copy(a_hbm, a_v)
    pltpu.sync_copy(z_hbm, sh_h.at[pl.ds(t * ZPT, ZPT)])
    _stage_finish(0, egs[0])

    plsc.subcore_barrier()

    def _pair(p, _):
        _stage_start(2 * p + 1, egs[1])
        _process_group(egs[0])

        @pl.when(p < NP - 1)
        def _():
            _stage_start(2 * p + 2, egs[0])

        _stage_finish(2 * p + 1, egs[1])
        _process_group(egs[1])

        @pl.when(p < NP - 1)
        def _():
            _stage_finish(2 * p + 2, egs[0])

        return 0

    lax.fori_loop(0, NP, _pair, 0)

    plsc.subcore_barrier()

    alpha = a_v[...]
    bvecs = [b_v[pl.ds(c * 16, 16)] for c in range(H // 16)]
    K2 = NPER - K

    pool_pairs = (((buf0, gsems[0]), (buf1, gsems[1])),
                  ((buf2, gsems[2]), (sbuf0, ssems[0])))

    def _pool_copies(g, q):
        (ba, sa), (bb, sb) = pool_pairs[q]
        return (pltpu.make_async_copy(sh_h.at[pl.ds(g * NPER, K)], ba, sa),
                pltpu.make_async_copy(sh_h.at[pl.ds(g * NPER + K, K2)],
                                      bb.at[pl.ds(0, K2)], sb))

    NPOOL = 7

    def _pool_start(gi):
        g = t + NS * gi

        @pl.when(g < B)
        def _():
            for cp in _pool_copies(g, gi % 2):
                cp.start()

    _pool_start(0)
    for gi in range(NPOOL):
        g = t + NS * gi
        if gi + 1 < NPOOL:
            _pool_start(gi + 1)

        @pl.when(g < B)
        def _():
            q = gi % 2
            (ba, _sa), (bb, _sb) = pool_pairs[q]
            for cp in _pool_copies(g, q):
                cp.wait()

            def _mk_acc(buf):
                def _acc(r, accs):
                    out = []
                    for c in range(H // 16):
                        sl = pl.ds(c * 16, 16)
                        v = buf[r, sl] + bvecs[c]
                        v = jnp.where(v >= 0.0, v, v * alpha)
                        out.append(accs[c] + v)
                    return tuple(out)
                return _acc

            zero8 = tuple(jnp.zeros((16,), jnp.float32)
                          for _ in range(H // 16))
            accs = lax.fori_loop(0, K, _mk_acc(ba), zero8)
            accs = lax.fori_loop(0, K2, _mk_acc(bb), accs)
            for c in range(H // 16):
                prow[pl.ds(c * 16, 16)] = accs[c] * (1.0 / NPER)
            pltpu.sync_copy(prow, out_hbm.at[half, g])


_SC_MESH = plsc.VectorSubcoreMesh(
    core_axis_name="c", subcore_axis_name="s", num_cores=NC, num_subcores=NS)

_sc_aggregate = pl.kernel(
    _sc_body,
    out_type=jax.ShapeDtypeStruct((NC, B, H), jnp.float32),
    mesh=_SC_MESH,
    compiler_params=pltpu.CompilerParams(needs_layout_passes=False),
    scratch_types=[
        pltpu.VMEM((G, K), jnp.int32),
        pltpu.VMEM((G, K), jnp.int32),
        pltpu.VMEM((G, K), jnp.float32),
        pltpu.VMEM((G, K), jnp.int32),
        pltpu.VMEM((G, K), jnp.int32),
        pltpu.VMEM((G, K), jnp.float32),
        pltpu.VMEM((K, H), jnp.float32),
        pltpu.VMEM((K, H), jnp.float32),
        pltpu.VMEM((K, H), jnp.float32),
        pltpu.VMEM((K, H), jnp.float32),
        pltpu.VMEM((K, H), jnp.float32),
        pltpu.VMEM((H,), jnp.float32),
        pltpu.VMEM((16,), jnp.float32),
        pltpu.VMEM((H,), jnp.float32),
        pltpu.SemaphoreType.DMA,
        pltpu.SemaphoreType.DMA,
        pltpu.SemaphoreType.DMA,
        pltpu.SemaphoreType.DMA,
        pltpu.SemaphoreType.DMA,
        pltpu.SemaphoreType.DMA,
        pltpu.SemaphoreType.DMA,
        pltpu.VMEM_SHARED((N, H), jnp.float32),
    ],
)


def _norm_body(parts_ref, aout_ref, pooled_ref, anch_ref):
    p0 = parts_ref[0]
    p1 = parts_ref[1]
    ss = (jnp.sum(p0 * p0, axis=1, keepdims=True)
          + jnp.sum(p1 * p1, axis=1, keepdims=True))
    d = jnp.maximum(jnp.sqrt(ss), 1e-12)
    pooled_ref[:, :H] = p0 / d
    pooled_ref[:, H:] = p1 / d
    a = aout_ref[...]
    da = jnp.maximum(jnp.sqrt(jnp.sum(a * a, axis=1, keepdims=True)), 1e-12)
    anch_ref[...] = a / da


def _phase_c(parts, anchor_out):
    return pl.pallas_call(
        _norm_body,
        out_shape=[
            jax.ShapeDtypeStruct((B, D), jnp.float32),
            jax.ShapeDtypeStruct((B, D), jnp.float32),
        ],
    )(parts, anchor_out)


def kernel(x, edge_index, edge_w, W, b, prelu_a):
    x = x.astype(jnp.float32)
    anchors = x.reshape(B, NPER, D)[:, 0, :]
    b2 = b.astype(jnp.float32).reshape(1, D)
    feat2, anchor_out = _phase_a(x, W.astype(jnp.float32), b2, anchors)

    src = edge_index[0].astype(jnp.int32)
    dst = edge_index[1].astype(jnp.int32)
    pad = EP - E
    zpad_i = jnp.zeros((pad,), jnp.int32)
    srcp = jnp.concatenate([src, zpad_i]).reshape(NS, CH, K)
    dstp = jnp.concatenate([dst, zpad_i]).reshape(NS, CH, K)
    wp = jnp.concatenate(
        [edge_w.astype(jnp.float32), jnp.zeros((pad,), jnp.float32)]
    ).reshape(NS, CH, K)
    bhalf = b.astype(jnp.float32).reshape(NC, H)
    a16 = jnp.broadcast_to(prelu_a.astype(jnp.float32), (16,))
    zrows = jnp.zeros((ZPT, H), jnp.float32)
    feat_flat = feat2.reshape(2 * N, H)

    parts = _sc_aggregate(feat_flat, srcp, dstp, wp, bhalf, a16, zrows)
    pooled_n, anchor_n = _phase_c(parts, anchor_out)
    return (pooled_n, anchor_n)


# speedup vs baseline: 1.5627x; 1.5627x over previous
import jax
import jax.numpy as jnp
from jax import lax
from jax.experimental import pallas as pl
from jax.experimental.pallas import tpu as pltpu
from jax.experimental.pallas import tpu_sc as plsc

N = 10000
B = 100
NPER = 100
D = 256
H = 128
E = 160000
NC = 2
NS = 16
K = 128
G = 8
NG = 10
NP = NG // 2
CH = G * NG
EPT = CH * K
EP = NS * EPT
ZPT = N // NS
RB = 2000

def _mm_body(x_ref, w_ref, b_ref, anch_ref, feat_ref, aout_ref):
    i = pl.program_id(0)
    xb = x_ref[...]
    row = lax.broadcasted_iota(jnp.int32, (RB, 1), 0) + i * RB
    xb = jnp.where(row % NPER == 0, 0.0, xb)
    f = jnp.dot(xb, w_ref[...], preferred_element_type=jnp.float32)
    feat_ref[0] = f[:, :H]
    feat_ref[1] = f[:, H:]

    @pl.when(i == 0)
    def _():
        aout_ref[...] = (
            jnp.dot(anch_ref[...], w_ref[...], preferred_element_type=jnp.float32)
            + b_ref[...]
        )


def _phase_a(x, W, b2, anchors):
    return pl.pallas_call(
        _mm_body,
        grid=(N // RB,),
        in_specs=[
            pl.BlockSpec((RB, D), lambda i: (i, 0)),
            pl.BlockSpec((D, D), lambda i: (0, 0)),
            pl.BlockSpec((1, D), lambda i: (0, 0)),
            pl.BlockSpec((B, D), lambda i: (0, 0)),
        ],
        out_specs=[
            pl.BlockSpec((2, RB, H), lambda i: (0, i, 0)),
            pl.BlockSpec((B, D), lambda i: (0, 0)),
        ],
        out_shape=[
            jax.ShapeDtypeStruct((2, N, H), jnp.float32),
            jax.ShapeDtypeStruct((B, D), jnp.float32),
        ],
    )(x, W, b2, anchors)


def _sc_body(feat_hbm, src_hbm, dst_hbm, w_hbm, b_hbm, a_hbm, z_hbm, out_hbm,
             es0, ed0, ew0, es1, ed1, ew1, buf0, buf1,
             b_v, a_v, prow,
             gsem0, gsem1, ssem0, ssem1,
             esem0, esem1, sh_h):
    half = lax.axis_index("c")
    t = lax.axis_index("s")
    bufs = (buf0, buf1)
    gsems = (gsem0, gsem1)
    ssems = (ssem0, ssem1)
    egs = ((es0, ed0, ew0, esem0), (es1, ed1, ew1, esem1))

    off = half * N

    def _stage_copies(gi, eg):
        es, ed, ew, esem = eg
        gsl = pl.ds(gi * G, G)
        return (pltpu.make_async_copy(src_hbm.at[t, gsl], es, esem),
                pltpu.make_async_copy(dst_hbm.at[t, gsl], ed, esem),
                pltpu.make_async_copy(w_hbm.at[t, gsl], ew, esem))

    def _stage_start(gi, eg):
        for cp in _stage_copies(gi, eg):
            cp.start()

    def _stage_finish(gi, eg):
        for cp in _stage_copies(gi, eg):
            cp.wait()
        es = eg[0]
        for j in range(G):
            for c in range(K // 16):
                sl = pl.ds(c * 16, 16)
                es[j, sl] = es[j, sl] + off

    def _gather(j, eg):
        p = j % 2
        return pltpu.make_async_copy(feat_hbm.at[eg[0].at[j]], bufs[p],
                                     gsems[p])

    def _scatter(j, eg):
        p = j % 2
        return pltpu.make_async_copy(bufs[p], sh_h.at[eg[1].at[j]], ssems[p])

    def _scale(j, eg):
        buf = bufs[j % 2]
        ew = eg[2]

        def _body(k, _):
            wk = plsc.load_gather(
                ew, [jnp.full((16,), j, jnp.int32),
                     jnp.full((16,), k, jnp.int32)])
            for c in range(H // 16):
                sl = pl.ds(c * 16, 16)
                buf[k, sl] = buf[k, sl] * wk
            return 0

        lax.fori_loop(0, K, _body, 0, unroll=4)

    def _process_group(eg):
        _gather(0, eg).start()
        for j in range(G):
            if j + 1 < G:
                if j >= 1:
                    _scatter(j - 1, eg).wait()
                _gather(j + 1, eg).start()
            _gather(j, eg).wait()
            _scale(j, eg)
            _scatter(j, eg).start(add=True)
        _scatter(G - 2, eg).wait()
        _scatter(G - 1, eg).wait()

    _stage_start(0, egs[0])
    pltpu.sync_copy(b_hbm.at[half], b_v)
    pltpu.sync_copy(a_hbm, a_v)
    pltpu.sync_copy(z_hbm, sh_h.at[pl.ds(t * ZPT, ZPT)])
    _stage_finish(0, egs[0])

    plsc.subcore_barrier()

    def _pair(p, _):
        _stage_start(2 * p + 1, egs[1])
        _process_group(egs[0])

        @pl.when(p < NP - 1)
        def _():
            _stage_start(2 * p + 2, egs[0])

        _stage_finish(2 * p + 1, egs[1])
        _process_group(egs[1])

        @pl.when(p < NP - 1)
        def _():
            _stage_finish(2 * p + 2, egs[0])

        return 0

    lax.fori_loop(0, NP, _pair, 0)

    plsc.subcore_barrier()

    alpha = a_v[...]
    bvecs = [b_v[pl.ds(c * 16, 16)] for c in range(H // 16)]

    def _pool_copy(g, q):
        return pltpu.make_async_copy(sh_h.at[pl.ds(g * NPER, NPER)],
                                     bufs[q].at[pl.ds(0, NPER)], gsems[q])

    NPOOL = 7

    def _pool_start(gi):
        g = t + NS * gi

        @pl.when(g < B)
        def _():
            _pool_copy(g, gi % 2).start()

    _pool_start(0)
    for gi in range(NPOOL):
        g = t + NS * gi
        if gi + 1 < NPOOL:
            _pool_start(gi + 1)

        @pl.when(g < B)
        def _():
            q = gi % 2
            _pool_copy(g, q).wait()
            buf = bufs[q]

            def _acc(r, accs):
                out = []
                for c in range(H // 16):
                    sl = pl.ds(c * 16, 16)
                    v = buf[r, sl] + bvecs[c]
                    v = jnp.where(v >= 0.0, v, v * alpha)
                    out.append(accs[c] + v)
                return tuple(out)

            accs = lax.fori_loop(
                0, NPER, _acc,
                tuple(jnp.zeros((16,), jnp.float32) for _ in range(H // 16)))
            for c in range(H // 16):
                prow[pl.ds(c * 16, 16)] = accs[c] * (1.0 / NPER)
            pltpu.sync_copy(prow, out_hbm.at[half, g])


_SC_MESH = plsc.VectorSubcoreMesh(
    core_axis_name="c", subcore_axis_name="s", num_cores=NC, num_subcores=NS)

_sc_aggregate = pl.kernel(
    _sc_body,
    out_type=jax.ShapeDtypeStruct((NC, B, H), jnp.float32),
    mesh=_SC_MESH,
    compiler_params=pltpu.CompilerParams(needs_layout_passes=False),
    scratch_types=[
        pltpu.VMEM((G, K), jnp.int32),
        pltpu.VMEM((G, K), jnp.int32),
        pltpu.VMEM((G, K), jnp.float32),
        pltpu.VMEM((G, K), jnp.int32),
        pltpu.VMEM((G, K), jnp.int32),
        pltpu.VMEM((G, K), jnp.float32),
        pltpu.VMEM((K, H), jnp.float32),
        pltpu.VMEM((K, H), jnp.float32),
        pltpu.VMEM((H,), jnp.float32),
        pltpu.VMEM((16,), jnp.float32),
        pltpu.VMEM((H,), jnp.float32),
        pltpu.SemaphoreType.DMA,
        pltpu.SemaphoreType.DMA,
        pltpu.SemaphoreType.DMA,
        pltpu.SemaphoreType.DMA,
        pltpu.SemaphoreType.DMA,
        pltpu.SemaphoreType.DMA,
        pltpu.VMEM_SHARED((N, H), jnp.float32),
    ],
)


def _norm_body(parts_ref, aout_ref, pooled_ref, anch_ref):
    p0 = parts_ref[0]
    p1 = parts_ref[1]
    ss = (jnp.sum(p0 * p0, axis=1, keepdims=True)
          + jnp.sum(p1 * p1, axis=1, keepdims=True))
    d = jnp.maximum(jnp.sqrt(ss), 1e-12)
    pooled_ref[:, :H] = p0 / d
    pooled_ref[:, H:] = p1 / d
    a = aout_ref[...]
    da = jnp.maximum(jnp.sqrt(jnp.sum(a * a, axis=1, keepdims=True)), 1e-12)
    anch_ref[...] = a / da


def _phase_c(parts, anchor_out):
    return pl.pallas_call(
        _norm_body,
        out_shape=[
            jax.ShapeDtypeStruct((B, D), jnp.float32),
            jax.ShapeDtypeStruct((B, D), jnp.float32),
        ],
    )(parts, anchor_out)


def kernel(x, edge_index, edge_w, W, b, prelu_a):
    x = x.astype(jnp.float32)
    W = W.astype(jnp.float32)
    anchors = x.reshape(B, NPER, D)[:, 0, :]
    b2 = b.astype(jnp.float32).reshape(1, D)
    feat2, anchor_out = _phase_a(x, W, b2, anchors)

    src = edge_index[0].astype(jnp.int32)
    dst = edge_index[1].astype(jnp.int32)
    pad = EP - E
    zpad_i = jnp.zeros((pad,), jnp.int32)
    srcp = jnp.concatenate([src, zpad_i]).reshape(NS, CH, K)
    dstp = jnp.concatenate([dst, zpad_i]).reshape(NS, CH, K)
    wp = jnp.concatenate(
        [edge_w.astype(jnp.float32), jnp.zeros((pad,), jnp.float32)]
    ).reshape(NS, CH, K)
    bhalf = b.astype(jnp.float32).reshape(NC, H)
    a16 = jnp.broadcast_to(prelu_a.astype(jnp.float32), (16,))
    zrows = jnp.zeros((ZPT, H), jnp.float32)
    feat_flat = feat2.reshape(2 * N, H)

    parts = _sc_aggregate(feat_flat, srcp, dstp, wp, bhalf, a16, zrows)
    pooled_n, anchor_n = _phase_c(parts, anchor_out)
    return (pooled_n, anchor_n)
